# Initial kernel scaffold; baseline (speedup 1.0000x reference)
import jax
import jax.numpy as jnp
from jax.experimental import pallas as pl


def kernel(item_embedding, attr_embedding, edge_index, inputs, W1, b1, W2, b2, Wc1, bc1, Wc2, bc2):
    raise NotImplementedError("write your pallas kernel here")


# speedup vs baseline: 10.3832x; 10.3832x over previous
import functools

import jax
import jax.numpy as jnp
from jax import lax
from jax.experimental import pallas as pl
from jax.experimental.pallas import tpu as pltpu
from jax.experimental.pallas import tpu_sc as plsc

D = 128
NC = 2
NS = 16
NW = NC * NS
CH = 64
NCH = 160
EPT = CH * NCH
E_PAD = EPT * NW
N_PAD = 10240
RB = 1024
GRID = N_PAD // RB


def _enc_body(x_ref, w1t_ref, b1_ref, w2t_ref, b2_ref, wc1t_ref, out_ref):
    x = x_ref[...]
    h = jnp.dot(x, w1t_ref[...], preferred_element_type=jnp.float32) + b1_ref[...]
    h = jnp.where(h > 0, h, jnp.exp(h) - 1.0)
    h = jnp.dot(h, w2t_ref[...], preferred_element_type=jnp.float32) + b2_ref[...]
    out_ref[...] = jnp.dot(h, wc1t_ref[...], preferred_element_type=jnp.float32)


def _encoder_conv1(x_pad, w1t, b1, w2t, b2, wc1t):
    wspec = pl.BlockSpec((D, D), lambda i: (0, 0))
    bspec = pl.BlockSpec((D,), lambda i: (0,))
    return pl.pallas_call(
        _enc_body,
        grid=(GRID,),
        in_specs=[pl.BlockSpec((RB, D), lambda i: (i, 0)),
                  wspec, bspec, wspec, bspec, wspec],
        out_specs=pl.BlockSpec((RB, D), lambda i: (i, 0)),
        out_shape=jax.ShapeDtypeStruct((N_PAD, D), jnp.float32),
    )(x_pad, w1t, b1, w2t, b2, wc1t)


def _scale_body(degp_ref, h1_ref, dinv_ref, g1_ref):
    deg = jnp.sum(degp_ref[...], axis=0) + 1.0
    dinv = lax.rsqrt(deg)
    dinv_ref[...] = dinv
    g1_ref[...] = h1_ref[...] * dinv[:, None]


def _scale(deg_part, h1):
    return pl.pallas_call(
        _scale_body,
        grid=(GRID,),
        in_specs=[pl.BlockSpec((NW, RB), lambda i: (0, i)),
                  pl.BlockSpec((RB, D), lambda i: (i, 0))],
        out_specs=[pl.BlockSpec((RB,), lambda i: (i,)),
                   pl.BlockSpec((RB, D), lambda i: (i, 0))],
        out_shape=[jax.ShapeDtypeStruct((N_PAD,), jnp.float32),
                   jax.ShapeDtypeStruct((N_PAD, D), jnp.float32)],
    )(deg_part, h1)


def _mid_body(dinv_ref, g1_ref, a0_ref, a1_ref, bc1_ref, wc2t_ref, g2_ref):
    dinv = dinv_ref[...][:, None]
    e1 = dinv * (g1_ref[...] + a0_ref[...] + a1_ref[...]) + bc1_ref[...]
    e1 = jnp.maximum(e1, 0.0)
    g2_ref[...] = jnp.dot(e1, wc2t_ref[...], preferred_element_type=jnp.float32) * dinv


def _mid(dinv, g1, a0, a1, bc1, wc2t):
    rspec = pl.BlockSpec((RB, D), lambda i: (i, 0))
    return pl.pallas_call(
        _mid_body,
        grid=(GRID,),
        in_specs=[pl.BlockSpec((RB,), lambda i: (i,)), rspec, rspec, rspec,
                  pl.BlockSpec((D,), lambda i: (0,)),
                  pl.BlockSpec((D, D), lambda i: (0, 0))],
        out_specs=rspec,
        out_shape=jax.ShapeDtypeStruct((N_PAD, D), jnp.float32),
    )(dinv, g1, a0, a1, bc1, wc2t)


def _fin_body(dinv_ref, g2_ref, a0_ref, a1_ref, bc2_ref, e2_ref):
    dinv = dinv_ref[...][:, None]
    e2_ref[...] = dinv * (g2_ref[...] + a0_ref[...] + a1_ref[...]) + bc2_ref[...]


def _fin(dinv, g2, a0, a1, bc2):
    rspec = pl.BlockSpec((RB, D), lambda i: (i, 0))
    return pl.pallas_call(
        _fin_body,
        grid=(GRID,),
        in_specs=[pl.BlockSpec((RB,), lambda i: (i,)), rspec, rspec, rspec,
                  pl.BlockSpec((D,), lambda i: (0,))],
        out_specs=rspec,
        out_shape=jax.ShapeDtypeStruct((N_PAD, D), jnp.float32),
    )(dinv, g2, a0, a1, bc2)


def _loss_body(p_ref, out_ref):
    B = 4096
    x = p_ref[0:B, :]
    y = p_ref[B:2 * B, :]
    xn = x / jnp.maximum(jnp.sqrt(jnp.sum(x * x, axis=1, keepdims=True)), 1e-12)
    yn = y / jnp.maximum(jnp.sqrt(jnp.sum(y * y, axis=1, keepdims=True)), 1e-12)
    d = xn - yn
    out_ref[0, 0] = jnp.sum(d * d) / B


def _loss(pairs):
    return pl.pallas_call(
        _loss_body,
        out_specs=pl.BlockSpec(memory_space=pltpu.SMEM),
        out_shape=jax.ShapeDtypeStruct((1, 1), jnp.float32),
    )(pairs)


_MESH = plsc.VectorSubcoreMesh(core_axis_name="c", subcore_axis_name="s",
                               num_cores=NC, num_subcores=NS)


def _deg_body(dst_hbm, out_hbm, dstv, degl):
    c = lax.axis_index("c")
    s = lax.axis_index("s")
    wid = c * NS + s
    pltpu.sync_copy(dst_hbm.at[wid], dstv)
    zero16 = jnp.zeros((16,), jnp.float32)

    def _z(i, _):
        degl[pl.ds(i * 16, 16)] = zero16
        return 0
    lax.fori_loop(0, N_PAD // 16, _z, 0)

    one16 = jnp.ones((16,), jnp.float32)

    def _acc(j, _):
        for k in range(CH // 16):
            idx = dstv[j, k * 16:(k + 1) * 16]
            plsc.addupdate_scatter(degl, [idx], one16)
        return 0
    lax.fori_loop(0, NCH, _acc, 0)
    pltpu.sync_copy(degl, out_hbm.at[wid])


_deg_kernel = functools.partial(
    pl.kernel,
    out_type=jax.ShapeDtypeStruct((NW, N_PAD), jnp.float32),
    mesh=_MESH,
    compiler_params=pltpu.CompilerParams(needs_layout_passes=False),
    scratch_types=[pltpu.VMEM((NCH, CH), jnp.int32),
                   pltpu.VMEM((N_PAD,), jnp.float32)],
)(_deg_body)


NCH2 = NCH // 2


def _agg_body(g_hbm, src_hbm, dst_hbm, out_hbm, srcv, dstv, buf, shared,
              sem0, sem1):
    c = lax.axis_index("c")
    s = lax.axis_index("s")
    wid = c * NS + s

    zero16 = jnp.zeros((16,), jnp.float32)

    def _zrow(j, _):
        for k in range(D // 16):
            buf[0, j, k * 16:(k + 1) * 16] = zero16
        return 0
    lax.fori_loop(0, CH, _zrow, 0)
    nstripe = N_PAD // (NS * CH)
    for i in range(nstripe):
        pltpu.sync_copy(buf.at[0], shared.at[pl.ds((s * nstripe + i) * CH, CH)])
    plsc.subcore_barrier()

    for p in range(2):
        pltpu.sync_copy(src_hbm.at[wid, pl.ds(p * NCH2, NCH2)], srcv)
        pltpu.sync_copy(dst_hbm.at[wid, pl.ds(p * NCH2, NCH2)], dstv)
        pltpu.async_copy(g_hbm.at[srcv.at[0]], buf.at[0], sem0)

        def _step(i, _):
            j = 2 * i
            pltpu.async_copy(g_hbm.at[srcv.at[j + 1]], buf.at[1], sem1)
            pltpu.make_async_copy(g_hbm.at[srcv.at[j]], buf.at[0], sem0).wait()
            pltpu.sync_copy(buf.at[0], shared.at[dstv.at[j]], add=True)

            @pl.when(j + 2 < NCH2)
            def _():
                pltpu.async_copy(g_hbm.at[srcv.at[j + 2]], buf.at[0], sem0)

            pltpu.make_async_copy(g_hbm.at[srcv.at[j + 1]], buf.at[1], sem1).wait()
            pltpu.sync_copy(buf.at[1], shared.at[dstv.at[j + 1]], add=True)
            return 0
        lax.fori_loop(0, NCH2 // 2, _step, 0)
    plsc.subcore_barrier()

    rows = N_PAD // NS
    pltpu.sync_copy(shared.at[pl.ds(s * rows, rows)],
                    out_hbm.at[c, pl.ds(s * rows, rows)])


_agg_kernel = functools.partial(
    pl.kernel,
    out_type=jax.ShapeDtypeStruct((NC, N_PAD, D), jnp.float32),
    mesh=_MESH,
    scratch_types=[pltpu.VMEM((NCH2, CH), jnp.int32),
                   pltpu.VMEM((NCH2, CH), jnp.int32),
                   pltpu.VMEM((2, CH, D), jnp.float32),
                   pltpu.VMEM_SHARED((N_PAD, D), jnp.float32),
                   pltpu.SemaphoreType.DMA,
                   pltpu.SemaphoreType.DMA],
)(_agg_body)


PCH = 128


def _pair_body(e_hbm, idx_hbm, out_hbm, idxv, bufv, sem):
    c = lax.axis_index("c")
    s = lax.axis_index("s")
    wid = c * NS + s
    pltpu.sync_copy(idx_hbm.at[wid], idxv)
    pltpu.async_copy(e_hbm.at[idxv.at[0]], bufv.at[pl.ds(0, PCH)], sem).wait()
    pltpu.async_copy(e_hbm.at[idxv.at[1]], bufv.at[pl.ds(PCH, PCH)], sem).wait()
    pltpu.sync_copy(bufv, out_hbm.at[pl.ds(wid * 2 * PCH, 2 * PCH)])


_pair_kernel = functools.partial(
    pl.kernel,
    out_type=jax.ShapeDtypeStruct((2 * 4096, D), jnp.float32),
    mesh=_MESH,
    scratch_types=[pltpu.VMEM((2, PCH), jnp.int32),
                   pltpu.VMEM((2 * PCH, D), jnp.float32),
                   pltpu.SemaphoreType.DMA],
)(_pair_body)


def kernel(item_embedding, attr_embedding, edge_index, inputs,
           W1, b1, W2, b2, Wc1, bc1, Wc2, bc2):
    n_items = item_embedding.shape[0]
    n_attrs = attr_embedding.shape[0]
    n = n_items + n_attrs
    e = edge_index.shape[1]

    x_pad = jnp.zeros((N_PAD, D), jnp.float32)
    x_pad = lax.dynamic_update_slice(x_pad, item_embedding, (0, 0))
    x_pad = lax.dynamic_update_slice(x_pad, attr_embedding, (n_items, 0))

    src = jnp.full((E_PAD,), 0, jnp.int32).at[:e].set(edge_index[0])
    dst = jnp.full((E_PAD,), N_PAD - 1, jnp.int32).at[:e].set(edge_index[1])
    src_t = src.reshape(NW, NCH, CH)
    dst_t = dst.reshape(NW, NCH, CH)

    pair_idx = jnp.concatenate([inputs[:, 0], inputs[:, 1]]).reshape(NW, 2, PCH)

    h1 = _encoder_conv1(x_pad, W1.T, b1, W2.T, b2, Wc1.T)
    deg_part = _deg_kernel(dst_t)
    dinv, g1 = _scale(deg_part, h1)

    acc1 = _agg_kernel(g1, src_t, dst_t)
    g2 = _mid(dinv, g1, acc1[0], acc1[1], bc1, Wc2.T)
    acc2 = _agg_kernel(g2, src_t, dst_t)
    e2 = _fin(dinv, g2, acc2[0], acc2[1], bc2)

    pairs = _pair_kernel(e2, pair_idx)
    loss = _loss(pairs)[0, 0]
    return (loss, e2[:n])
